# baseline (speedup 1.0000x reference)
import functools

import jax
import jax.numpy as jnp
from jax import lax
from jax.experimental import pallas as pl
from jax.experimental.pallas import tpu as pltpu
from jax.experimental.pallas import tpu_sc as plsc

N = 10000
E = 160000
D_IN = 256
HID = 32
OUT = 2

NC = 2
NS = 16
NW = NC * NS

C = 128
EROWS = E // C
CH_MAX = 40
RS = 640
RS_LAST = N - 15 * RS

_sc_mesh = plsc.VectorSubcoreMesh(
    core_axis_name="c", subcore_axis_name="s", num_cores=NC, num_subcores=NS
)
_sc_params = pltpu.CompilerParams(
    use_tc_tiling_on_sc=False, needs_layout_passes=False
)


def _worker_chunks(w):
    base = 39 * w + jnp.minimum(w, 2)
    n = jnp.where(w < 2, 40, 39)
    return base, n


def _stripe(s):
    return s * RS, jnp.where(s == NS - 1, RS_LAST, RS)


def _stage_idx(idx_hbm, idx_v, base, w):
    @pl.when(w < 2)
    def _():
        pltpu.sync_copy(idx_hbm.at[pl.ds(base, 40)], idx_v)

    @pl.when(w >= 2)
    def _():
        pltpu.sync_copy(idx_hbm.at[pl.ds(base, 39)], idx_v.at[pl.ds(0, 39)])


def _stage_rows(src, dst, s, src0, dst0):
    @pl.when(s < NS - 1)
    def _():
        pltpu.sync_copy(src.at[pl.ds(src0, RS)], dst.at[pl.ds(dst0, RS)])

    @pl.when(s == NS - 1)
    def _():
        pltpu.sync_copy(src.at[pl.ds(src0, RS_LAST)],
                        dst.at[pl.ds(dst0, RS_LAST)])


def _rsqrt16(d):
    i = plsc.bitcast(d, jnp.int32)
    i = jnp.int32(0x5F3759DF) - lax.shift_right_logical(i, 1)
    y = plsc.bitcast(i, jnp.float32)
    for _ in range(3):
        y = y * (1.5 - 0.5 * d * y * y)
    return y


@functools.partial(
    pl.kernel,
    out_type=jax.ShapeDtypeStruct((NW * RS,), jnp.float32),
    mesh=_sc_mesh,
    scratch_types=[
        pltpu.VMEM((CH_MAX, C), jnp.int32),
        pltpu.VMEM((C,), jnp.float32),
        pltpu.VMEM_SHARED((N,), jnp.float32),
    ],
    compiler_params=_sc_params,
)
def _sc_degree(dst_hbm, zeros_hbm, out_hbm, dst_v, ones_v, deg_sh):
    c = lax.axis_index("c")
    s = lax.axis_index("s")
    w = c * NS + s
    base, nch = _worker_chunks(w)
    row0, _ = _stripe(s)
    _stage_rows(zeros_hbm, deg_sh, s, row0, row0)
    _stage_idx(dst_hbm, dst_v, base, w)
    for i in range(C // 16):
        ones_v[pl.ds(i * 16, 16)] = jnp.ones((16,), jnp.float32)
    plsc.subcore_barrier()

    @pl.loop(0, nch)
    def _(j):
        pltpu.sync_copy(ones_v, deg_sh.at[dst_v.at[j]], add=True)

    plsc.subcore_barrier()
    _stage_rows(deg_sh, out_hbm, s, row0, w * RS)


def _scale_rows(buf_v, norm_v, n16):
    @pl.loop(0, n16)
    def _(i):
        r0 = i * 16
        nvec = norm_v[pl.ds(r0, 16)]
        for b in range(16):
            nv = jnp.full((16,), nvec[b], jnp.float32)
            buf_v[r0 + b, pl.ds(0, 16)] = buf_v[r0 + b, pl.ds(0, 16)] * nv
            buf_v[r0 + b, pl.ds(16, 16)] = buf_v[r0 + b, pl.ds(16, 16)] * nv


def _agg_common(src_v, dst_v, rows_a, rows_b, g_sh, agg_sh, ga, gb, sa, sb,
                nch):
    npair = nch // 2

    pltpu.async_copy(g_sh.at[src_v.at[0]], rows_a, ga)

    @pl.loop(0, npair)
    def _(jj):
        j = 2 * jj
        pltpu.make_async_copy(g_sh.at[src_v.at[j]], rows_a, ga).wait()
        pltpu.async_copy(rows_a, agg_sh.at[dst_v.at[j]], sa, add=True)

        @pl.when(jj > 0)
        def _():
            pltpu.make_async_copy(rows_b, agg_sh.at[dst_v.at[j]], sb).wait()

        pltpu.async_copy(g_sh.at[src_v.at[j + 1]], rows_b, gb)
        pltpu.make_async_copy(g_sh.at[src_v.at[j + 1]], rows_b, gb).wait()
        pltpu.async_copy(rows_b, agg_sh.at[dst_v.at[j + 1]], sb, add=True)

        @pl.when(j + 2 < nch)
        def _():
            pltpu.make_async_copy(rows_a, agg_sh.at[dst_v.at[j]], sa).wait()
            pltpu.async_copy(g_sh.at[src_v.at[j + 2]], rows_a, ga)

    @pl.when(nch % 2 == 1)
    def _():
        pltpu.make_async_copy(g_sh.at[src_v.at[0]], rows_a, ga).wait()
        pltpu.async_copy(rows_a, agg_sh.at[dst_v.at[nch - 1]], sa, add=True)

    pltpu.make_async_copy(rows_a, agg_sh.at[dst_v.at[0]], sa).wait()
    pltpu.make_async_copy(rows_b, agg_sh.at[dst_v.at[0]], sb).wait()


def _emit_y(c, s, norm_v, abuf_v, gbuf_v, agg_sh, y_hbm, n16):
    cf = jnp.where(c == 1, 1.0, 0.0).astype(jnp.float32)
    cv = jnp.full((16,), cf, jnp.float32)
    row0, _ = _stripe(s)
    _stage_rows(agg_sh, abuf_v, s, row0, 0)

    @pl.loop(0, n16)
    def _(i):
        r0 = i * 16
        nvec = norm_v[pl.ds(r0, 16)]
        for b in range(16):
            nv = jnp.full((16,), nvec[b], jnp.float32)
            a0 = abuf_v[r0 + b, pl.ds(0, 16)] + gbuf_v[r0 + b, pl.ds(0, 16)] * cv
            a1 = abuf_v[r0 + b, pl.ds(16, 16)] + gbuf_v[r0 + b, pl.ds(16, 16)] * cv
            abuf_v[r0 + b, pl.ds(0, 16)] = a0 * nv
            abuf_v[r0 + b, pl.ds(16, 16)] = a1 * nv

    _stage_rows(abuf_v, y_hbm.at[c], s, 0, row0)


_agg_scratch = [
    pltpu.VMEM((CH_MAX, C), jnp.int32),
    pltpu.VMEM((CH_MAX, C), jnp.int32),
    pltpu.VMEM((C, HID), jnp.float32),
    pltpu.VMEM((C, HID), jnp.float32),
    pltpu.VMEM((RS,), jnp.float32),
    pltpu.VMEM((RS, HID), jnp.float32),
    pltpu.VMEM((RS, HID), jnp.float32),
    pltpu.VMEM_SHARED((N, HID), jnp.float32),
    pltpu.VMEM_SHARED((N, HID), jnp.float32),
    pltpu.SemaphoreType.DMA,
    pltpu.SemaphoreType.DMA,
    pltpu.SemaphoreType.DMA,
    pltpu.SemaphoreType.DMA,
]


@functools.partial(
    pl.kernel,
    out_type=(jax.ShapeDtypeStruct((NC, N, HID), jnp.float32),
              jax.ShapeDtypeStruct((NS * RS,), jnp.float32)),
    mesh=_sc_mesh,
    scratch_types=[pltpu.VMEM((2 * RS,), jnp.float32)] + _agg_scratch,
    compiler_params=_sc_params,
)
def _sc_agg1(h_hbm, deg_hbm, src_hbm, dst_hbm, zeros_hbm,
             y_hbm, norm_hbm,
             deg_v, src_v, dst_v, rows_a, rows_b, norm_v, gbuf_v, abuf_v,
             g_sh, agg_sh, ga, gb, sa, sb):
    c = lax.axis_index("c")
    s = lax.axis_index("s")
    w = c * NS + s
    base, nch = _worker_chunks(w)
    row0, nrows = _stripe(s)
    n16 = nrows // 16
    pltpu.sync_copy(deg_hbm.at[pl.ds(s * RS, RS)], deg_v.at[pl.ds(0, RS)])
    pltpu.sync_copy(deg_hbm.at[pl.ds((NS + s) * RS, RS)],
                    deg_v.at[pl.ds(RS, RS)])
    _stage_rows(h_hbm, gbuf_v, s, row0, 0)
    _stage_rows(zeros_hbm, agg_sh, s, row0, row0)
    _stage_idx(src_hbm, src_v, base, w)
    _stage_idx(dst_hbm, dst_v, base, w)

    @pl.loop(0, n16)
    def _(i):
        d = deg_v[pl.ds(i * 16, 16)] + deg_v[pl.ds(RS + i * 16, 16)] + 1.0
        norm_v[pl.ds(i * 16, 16)] = _rsqrt16(d)

    _scale_rows(gbuf_v, norm_v, n16)
    _stage_rows(gbuf_v, g_sh, s, 0, row0)

    @pl.when(c == 0)
    def _():
        pltpu.sync_copy(norm_v, norm_hbm.at[pl.ds(s * RS, RS)])

    plsc.subcore_barrier()
    _agg_common(src_v, dst_v, rows_a, rows_b, g_sh, agg_sh, ga, gb, sa, sb,
                nch)
    plsc.subcore_barrier()
    _emit_y(c, s, norm_v, abuf_v, gbuf_v, agg_sh, y_hbm, n16)


@functools.partial(
    pl.kernel,
    out_type=jax.ShapeDtypeStruct((NC, N, HID), jnp.float32),
    mesh=_sc_mesh,
    scratch_types=_agg_scratch,
    compiler_params=_sc_params,
)
def _sc_agg2(h_hbm, norm_in_hbm, src_hbm, dst_hbm, zeros_hbm,
             y_hbm,
             src_v, dst_v, rows_a, rows_b, norm_v, gbuf_v, abuf_v,
             g_sh, agg_sh, ga, gb, sa, sb):
    c = lax.axis_index("c")
    s = lax.axis_index("s")
    w = c * NS + s
    base, nch = _worker_chunks(w)
    row0, nrows = _stripe(s)
    n16 = nrows // 16
    pltpu.sync_copy(norm_in_hbm.at[pl.ds(s * RS, RS)], norm_v)
    _stage_rows(h_hbm, gbuf_v, s, row0, 0)
    _stage_rows(zeros_hbm, agg_sh, s, row0, row0)
    _stage_idx(src_hbm, src_v, base, w)
    _stage_idx(dst_hbm, dst_v, base, w)
    _scale_rows(gbuf_v, norm_v, n16)
    _stage_rows(gbuf_v, g_sh, s, 0, row0)
    plsc.subcore_barrier()
    _agg_common(src_v, dst_v, rows_a, rows_b, g_sh, agg_sh, ga, gb, sa, sb,
                nch)
    plsc.subcore_barrier()
    _emit_y(c, s, norm_v, abuf_v, gbuf_v, agg_sh, y_hbm, n16)


_BLK = 2000


def _k1_body(x_ref, w1_ref, h1_ref):
    h1_ref[...] = jnp.dot(x_ref[...], w1_ref[...],
                          preferred_element_type=jnp.float32)


def _k3_body(y_ref, b1_ref, w2_ref, h2_ref):
    h = jnp.maximum(y_ref[0] + y_ref[1] + b1_ref[...], 0.0)
    h2_ref[...] = jnp.dot(h, w2_ref[...], preferred_element_type=jnp.float32)


def _k4_body(y_ref, b2_ref, w3_ref, b3_ref, out_ref):
    h = jnp.maximum(y_ref[0] + y_ref[1] + b2_ref[...], 0.0)
    logits = jnp.dot(h, w3_ref[...],
                     preferred_element_type=jnp.float32) + b3_ref[...]
    m = jnp.max(logits, axis=-1, keepdims=True)
    e = jnp.exp(logits - m)
    out_ref[...] = e / jnp.sum(e, axis=-1, keepdims=True)


def _tc_k1(x, W1):
    return pl.pallas_call(
        _k1_body,
        grid=(N // _BLK,),
        in_specs=[
            pl.BlockSpec((_BLK, D_IN), lambda i: (i, 0)),
            pl.BlockSpec((D_IN, HID), lambda i: (0, 0)),
        ],
        out_specs=pl.BlockSpec((_BLK, HID), lambda i: (i, 0)),
        out_shape=jax.ShapeDtypeStruct((N, HID), jnp.float32),
    )(x, W1)


def _tc_k3(y, b1, W2):
    return pl.pallas_call(
        _k3_body,
        grid=(N // _BLK,),
        in_specs=[
            pl.BlockSpec((NC, _BLK, HID), lambda i: (0, i, 0)),
            pl.BlockSpec((1, HID), lambda i: (0, 0)),
            pl.BlockSpec((HID, HID), lambda i: (0, 0)),
        ],
        out_specs=pl.BlockSpec((_BLK, HID), lambda i: (i, 0)),
        out_shape=jax.ShapeDtypeStruct((N, HID), jnp.float32),
    )(y, b1, W2)


def _tc_k4(y, b2, W3, b3):
    return pl.pallas_call(
        _k4_body,
        grid=(N // _BLK,),
        in_specs=[
            pl.BlockSpec((NC, _BLK, HID), lambda i: (0, i, 0)),
            pl.BlockSpec((1, HID), lambda i: (0, 0)),
            pl.BlockSpec((HID, OUT), lambda i: (0, 0)),
            pl.BlockSpec((1, OUT), lambda i: (0, 0)),
        ],
        out_specs=pl.BlockSpec((_BLK, OUT), lambda i: (i, 0)),
        out_shape=jax.ShapeDtypeStruct((N, OUT), jnp.float32),
    )(y, b2, W3, b3)


@jax.jit
def kernel(x, edge_index, W1, b1, W2, b2, W3, b3):
    src2 = edge_index[0].reshape(EROWS, C)
    dst2 = edge_index[1].reshape(EROWS, C)
    zeros_col = jnp.zeros((N,), jnp.float32)
    zeros_feat = jnp.zeros((N, HID), jnp.float32)

    deg = _sc_degree(dst2, zeros_col)
    h1 = _tc_k1(x, W1)
    y1, norm = _sc_agg1(h1, deg, src2, dst2, zeros_feat)
    h2 = _tc_k3(y1, b1.reshape(1, HID), W2)
    y2 = _sc_agg2(h2, norm, src2, dst2, zeros_feat)
    probs = _tc_k4(y2, b2.reshape(1, HID), W3, b3.reshape(1, OUT))
    return probs


# speedup vs baseline: 1.0209x; 1.0209x over previous
import functools

import jax
import jax.numpy as jnp
from jax import lax
from jax.experimental import pallas as pl
from jax.experimental.pallas import tpu as pltpu
from jax.experimental.pallas import tpu_sc as plsc

N = 10000
E = 160000
D_IN = 256
HID = 32
OUT = 2

NC = 2
NS = 16
NW = NC * NS

C = 128
EW = E // NW
CPW = 40
EWP = CPW * C
NPAD = N + 8
RS = 640
RS_LAST = N - 15 * RS

_sc_mesh = plsc.VectorSubcoreMesh(
    core_axis_name="c", subcore_axis_name="s", num_cores=NC, num_subcores=NS
)
_sc_params = pltpu.CompilerParams(
    use_tc_tiling_on_sc=False, needs_layout_passes=False
)


def _stripe(s):
    return s * RS, jnp.where(s == NS - 1, RS_LAST, RS)


def _stage_idx(ei_hbm, row, idx_v, w):
    pltpu.sync_copy(ei_hbm.at[row, pl.ds(EW * w, EW)], idx_v.at[pl.ds(0, EW)])
    dummy = jnp.full((16,), N, jnp.int32)
    for k in range(7):
        idx_v[pl.ds(EW + 16 * k, 16)] = dummy
    idx_v[pl.ds(EWP - 16, 16)] = dummy


def _stage_rows(src, dst, s, src0, dst0):
    @pl.when(s < NS - 1)
    def _():
        pltpu.sync_copy(src.at[pl.ds(src0, RS)], dst.at[pl.ds(dst0, RS)])

    @pl.when(s == NS - 1)
    def _():
        pltpu.sync_copy(src.at[pl.ds(src0, RS_LAST)],
                        dst.at[pl.ds(dst0, RS_LAST)])


def _rsqrt16(d):
    i = plsc.bitcast(d, jnp.int32)
    i = jnp.int32(0x5F3759DF) - lax.shift_right_logical(i, 1)
    y = plsc.bitcast(i, jnp.float32)
    for _ in range(3):
        y = y * (1.5 - 0.5 * d * y * y)
    return y


@functools.partial(
    pl.kernel,
    out_type=jax.ShapeDtypeStruct((NW * RS,), jnp.float32),
    mesh=_sc_mesh,
    scratch_types=[
        pltpu.VMEM((EWP,), jnp.int32),
        pltpu.VMEM((C,), jnp.float32),
        pltpu.VMEM_SHARED((NPAD,), jnp.float32),
    ],
    compiler_params=_sc_params,
)
def _sc_degree(ei_hbm, zeros_hbm, out_hbm, dst_v, ones_v, deg_sh):
    c = lax.axis_index("c")
    s = lax.axis_index("s")
    w = c * NS + s
    row0, _ = _stripe(s)
    _stage_rows(zeros_hbm, deg_sh, s, row0, row0)
    _stage_idx(ei_hbm, 1, dst_v, w)
    for i in range(C // 16):
        ones_v[pl.ds(i * 16, 16)] = jnp.ones((16,), jnp.float32)
    plsc.subcore_barrier()

    @pl.loop(0, CPW)
    def _(j):
        pltpu.sync_copy(ones_v, deg_sh.at[dst_v.at[pl.ds(j * C, C)]], add=True)

    plsc.subcore_barrier()
    _stage_rows(deg_sh, out_hbm, s, row0, w * RS)


def _scale_rows(buf_v, norm_v, n16):
    @pl.loop(0, n16)
    def _(i):
        r0 = i * 16
        nvec = norm_v[pl.ds(r0, 16)]
        for b in range(16):
            nv = jnp.full((16,), nvec[b], jnp.float32)
            buf_v[r0 + b, pl.ds(0, 16)] = buf_v[r0 + b, pl.ds(0, 16)] * nv
            buf_v[r0 + b, pl.ds(16, 16)] = buf_v[r0 + b, pl.ds(16, 16)] * nv


def _agg_common(src_v, dst_v, rows_a, rows_b, g_sh, agg_sh, ga, gb, sa, sb):
    HALF = CPW // 2

    def src_at(j):
        return src_v.at[pl.ds(j * C, C)]

    def dst_at(j):
        return dst_v.at[pl.ds(j * C, C)]

    pltpu.async_copy(g_sh.at[src_at(0)], rows_a, ga)

    @pl.loop(0, HALF)
    def _(jj):
        j = 2 * jj
        pltpu.make_async_copy(g_sh.at[src_at(j)], rows_a, ga).wait()
        pltpu.async_copy(rows_a, agg_sh.at[dst_at(j)], sa, add=True)

        @pl.when(jj > 0)
        def _():
            pltpu.make_async_copy(rows_b, agg_sh.at[dst_at(j)], sb).wait()

        pltpu.async_copy(g_sh.at[src_at(j + 1)], rows_b, gb)
        pltpu.make_async_copy(g_sh.at[src_at(j + 1)], rows_b, gb).wait()
        pltpu.async_copy(rows_b, agg_sh.at[dst_at(j + 1)], sb, add=True)

        @pl.when(jj < HALF - 1)
        def _():
            pltpu.make_async_copy(rows_a, agg_sh.at[dst_at(j)], sa).wait()
            pltpu.async_copy(g_sh.at[src_at(j + 2)], rows_a, ga)

    pltpu.make_async_copy(rows_a, agg_sh.at[dst_at(0)], sa).wait()
    pltpu.make_async_copy(rows_b, agg_sh.at[dst_at(0)], sb).wait()


def _emit_y(c, s, norm_v, abuf_v, gbuf_v, agg_sh, y_hbm, n16):
    cf = jnp.where(c == 1, 1.0, 0.0).astype(jnp.float32)
    cv = jnp.full((16,), cf, jnp.float32)
    row0, _ = _stripe(s)
    _stage_rows(agg_sh, abuf_v, s, row0, 0)

    @pl.loop(0, n16)
    def _(i):
        r0 = i * 16
        nvec = norm_v[pl.ds(r0, 16)]
        for b in range(16):
            nv = jnp.full((16,), nvec[b], jnp.float32)
            a0 = abuf_v[r0 + b, pl.ds(0, 16)] + gbuf_v[r0 + b, pl.ds(0, 16)] * cv
            a1 = abuf_v[r0 + b, pl.ds(16, 16)] + gbuf_v[r0 + b, pl.ds(16, 16)] * cv
            abuf_v[r0 + b, pl.ds(0, 16)] = a0 * nv
            abuf_v[r0 + b, pl.ds(16, 16)] = a1 * nv

    _stage_rows(abuf_v, y_hbm.at[c], s, 0, row0)


_agg_scratch = [
    pltpu.VMEM((EWP,), jnp.int32),
    pltpu.VMEM((EWP,), jnp.int32),
    pltpu.VMEM((C, HID), jnp.float32),
    pltpu.VMEM((C, HID), jnp.float32),
    pltpu.VMEM((RS,), jnp.float32),
    pltpu.VMEM((RS, HID), jnp.float32),
    pltpu.VMEM((RS, HID), jnp.float32),
    pltpu.VMEM_SHARED((NPAD, HID), jnp.float32),
    pltpu.VMEM_SHARED((NPAD, HID), jnp.float32),
    pltpu.SemaphoreType.DMA,
    pltpu.SemaphoreType.DMA,
    pltpu.SemaphoreType.DMA,
    pltpu.SemaphoreType.DMA,
]


@functools.partial(
    pl.kernel,
    out_type=(jax.ShapeDtypeStruct((NC, N, HID), jnp.float32),
              jax.ShapeDtypeStruct((NS * RS,), jnp.float32)),
    mesh=_sc_mesh,
    scratch_types=[pltpu.VMEM((2 * RS,), jnp.float32)] + _agg_scratch,
    compiler_params=_sc_params,
)
def _sc_agg1(h_hbm, deg_hbm, ei_hbm, zeros_hbm,
             y_hbm, norm_hbm,
             deg_v, src_v, dst_v, rows_a, rows_b, norm_v, gbuf_v, abuf_v,
             g_sh, agg_sh, ga, gb, sa, sb):
    c = lax.axis_index("c")
    s = lax.axis_index("s")
    w = c * NS + s
    row0, nrows = _stripe(s)
    n16 = nrows // 16
    pltpu.sync_copy(deg_hbm.at[pl.ds(s * RS, RS)], deg_v.at[pl.ds(0, RS)])
    pltpu.sync_copy(deg_hbm.at[pl.ds((NS + s) * RS, RS)],
                    deg_v.at[pl.ds(RS, RS)])
    _stage_rows(h_hbm, gbuf_v, s, row0, 0)
    _stage_rows(zeros_hbm, agg_sh, s, row0, row0)
    _stage_idx(ei_hbm, 0, src_v, w)
    _stage_idx(ei_hbm, 1, dst_v, w)

    @pl.loop(0, n16)
    def _(i):
        d = deg_v[pl.ds(i * 16, 16)] + deg_v[pl.ds(RS + i * 16, 16)] + 1.0
        norm_v[pl.ds(i * 16, 16)] = _rsqrt16(d)

    _scale_rows(gbuf_v, norm_v, n16)
    _stage_rows(gbuf_v, g_sh, s, 0, row0)

    @pl.when(s == 0)
    def _():
        pltpu.sync_copy(zeros_hbm.at[pl.ds(0, 8)], g_sh.at[pl.ds(N, 8)])

    @pl.when(c == 0)
    def _():
        pltpu.sync_copy(norm_v, norm_hbm.at[pl.ds(s * RS, RS)])

    plsc.subcore_barrier()
    _agg_common(src_v, dst_v, rows_a, rows_b, g_sh, agg_sh, ga, gb, sa, sb)
    plsc.subcore_barrier()
    _emit_y(c, s, norm_v, abuf_v, gbuf_v, agg_sh, y_hbm, n16)


@functools.partial(
    pl.kernel,
    out_type=jax.ShapeDtypeStruct((NC, N, HID), jnp.float32),
    mesh=_sc_mesh,
    scratch_types=_agg_scratch,
    compiler_params=_sc_params,
)
def _sc_agg2(h_hbm, norm_in_hbm, ei_hbm, zeros_hbm,
             y_hbm,
             src_v, dst_v, rows_a, rows_b, norm_v, gbuf_v, abuf_v,
             g_sh, agg_sh, ga, gb, sa, sb):
    c = lax.axis_index("c")
    s = lax.axis_index("s")
    w = c * NS + s
    row0, nrows = _stripe(s)
    n16 = nrows // 16
    pltpu.sync_copy(norm_in_hbm.at[pl.ds(s * RS, RS)], norm_v)
    _stage_rows(h_hbm, gbuf_v, s, row0, 0)
    _stage_rows(zeros_hbm, agg_sh, s, row0, row0)
    _stage_idx(ei_hbm, 0, src_v, w)
    _stage_idx(ei_hbm, 1, dst_v, w)
    _scale_rows(gbuf_v, norm_v, n16)
    _stage_rows(gbuf_v, g_sh, s, 0, row0)

    @pl.when(s == 0)
    def _():
        pltpu.sync_copy(zeros_hbm.at[pl.ds(0, 8)], g_sh.at[pl.ds(N, 8)])
    plsc.subcore_barrier()
    _agg_common(src_v, dst_v, rows_a, rows_b, g_sh, agg_sh, ga, gb, sa, sb)
    plsc.subcore_barrier()
    _emit_y(c, s, norm_v, abuf_v, gbuf_v, agg_sh, y_hbm, n16)


_BLK = 2000


def _k1_body(x_ref, w1_ref, h1_ref):
    h1_ref[...] = jnp.dot(x_ref[...], w1_ref[...],
                          preferred_element_type=jnp.float32)


def _k3_body(y_ref, b1_ref, w2_ref, h2_ref):
    h = jnp.maximum(y_ref[0] + y_ref[1] + b1_ref[...], 0.0)
    h2_ref[...] = jnp.dot(h, w2_ref[...], preferred_element_type=jnp.float32)


def _k4_body(y_ref, b2_ref, w3_ref, b3_ref, out_ref):
    h = jnp.maximum(y_ref[0] + y_ref[1] + b2_ref[...], 0.0)
    logits = jnp.dot(h, w3_ref[...],
                     preferred_element_type=jnp.float32) + b3_ref[...]
    m = jnp.max(logits, axis=-1, keepdims=True)
    e = jnp.exp(logits - m)
    out_ref[...] = e / jnp.sum(e, axis=-1, keepdims=True)


def _tc_k1(x, W1):
    return pl.pallas_call(
        _k1_body,
        grid=(N // _BLK,),
        in_specs=[
            pl.BlockSpec((_BLK, D_IN), lambda i: (i, 0)),
            pl.BlockSpec((D_IN, HID), lambda i: (0, 0)),
        ],
        out_specs=pl.BlockSpec((_BLK, HID), lambda i: (i, 0)),
        out_shape=jax.ShapeDtypeStruct((N, HID), jnp.float32),
    )(x, W1)


def _tc_k3(y, b1, W2):
    return pl.pallas_call(
        _k3_body,
        grid=(N // _BLK,),
        in_specs=[
            pl.BlockSpec((NC, _BLK, HID), lambda i: (0, i, 0)),
            pl.BlockSpec((1, HID), lambda i: (0, 0)),
            pl.BlockSpec((HID, HID), lambda i: (0, 0)),
        ],
        out_specs=pl.BlockSpec((_BLK, HID), lambda i: (i, 0)),
        out_shape=jax.ShapeDtypeStruct((N, HID), jnp.float32),
    )(y, b1, W2)


def _tc_k4(y, b2, W3, b3):
    return pl.pallas_call(
        _k4_body,
        grid=(N // _BLK,),
        in_specs=[
            pl.BlockSpec((NC, _BLK, HID), lambda i: (0, i, 0)),
            pl.BlockSpec((1, HID), lambda i: (0, 0)),
            pl.BlockSpec((HID, OUT), lambda i: (0, 0)),
            pl.BlockSpec((1, OUT), lambda i: (0, 0)),
        ],
        out_specs=pl.BlockSpec((_BLK, OUT), lambda i: (i, 0)),
        out_shape=jax.ShapeDtypeStruct((N, OUT), jnp.float32),
    )(y, b2, W3, b3)


@jax.jit
def kernel(x, edge_index, W1, b1, W2, b2, W3, b3):
    zeros_col = jnp.zeros((N,), jnp.float32)
    zeros_feat = jnp.zeros((N, HID), jnp.float32)

    deg = _sc_degree(edge_index, zeros_col)
    h1 = _tc_k1(x, W1)
    y1, norm = _sc_agg1(h1, deg, edge_index, zeros_feat)
    h2 = _tc_k3(y1, b1.reshape(1, HID), W2)
    y2 = _sc_agg2(h2, norm, edge_index, zeros_feat)
    probs = _tc_k4(y2, b2.reshape(1, HID), W3, b3.reshape(1, OUT))
    return probs
